# baseline (speedup 1.0000x reference)
import jax
import jax.numpy as jnp
from jax import lax
from jax.experimental import pallas as pl
from jax.experimental.pallas import tpu as pltpu
from jax.experimental.pallas import tpu_sc as plsc

_B = 16
_S = 2048
_D = 256
_TOTAL = 16384
_HALF = _S // 2
_C = 128
_NCHUNK = _HALF // _C
_L = 16
_NS = 16
_ZROWS = _C // _NS


def _body(flat, cu, out, cu_v, zsh, buf0, buf1, buf2, gidx, sidx,
          sem_c, sem_z, sem_i0, sem_i1, sem_i2, sem_o0, sem_o1, sem_o2):
    cid = lax.axis_index("c")
    sid = lax.axis_index("s")
    wid = cid * 16 + sid
    b = wid // 2
    h = wid % 2

    pltpu.async_copy(cu, cu_v.at[pl.ds(0, _B + 1)], sem_c)

    def _zero(i, _):
        buf0[i // (_D // _L), pl.ds((i % (_D // _L)) * _L, _L)] = (
            jnp.zeros((_L,), jnp.float32))
        return 0
    lax.fori_loop(0, _ZROWS * (_D // _L), _zero, 0, unroll=8)
    pltpu.sync_copy(buf0.at[pl.ds(0, _ZROWS), :],
                    zsh.at[pl.ds(sid * _ZROWS, _ZROWS), :])
    plsc.subcore_barrier()

    pltpu.make_async_copy(cu, cu_v.at[pl.ds(0, _B + 1)], sem_c).wait()
    start = cu_v[pl.ds(b, 16)][0]
    ln = cu_v[pl.ds(b + 1, 16)][0] - start
    t = jnp.clip(ln - h * _HALF, 0, _HALF)
    base = b * _S + h * _HALF
    floor_t = (t // _C) * _C
    nz = (_HALF - floor_t) // _C
    nd = (t + _C - 1) // _C

    lanes = lax.iota(jnp.int32, _L)

    def _off(i):
        return jnp.minimum(i * _C, t - _C)

    def _fill_idx(ref, slot, first_row):
        for k in range(_C // _L):
            ref[slot, pl.ds(k * _L, _L)] = first_row + k * _L + lanes

    def _gather(i, slot, isem):
        _fill_idx(gidx, slot, start + h * _HALF + _off(i))
        pltpu.async_copy(flat.at[gidx.at[slot]], (buf0, buf1, buf2)[slot], isem)

    bufs = (buf0, buf1, buf2)
    isems = (sem_i0, sem_i1, sem_i2)
    osems = (sem_o0, sem_o1, sem_o2)

    @pl.when(0 < nd)
    def _():
        _gather(0, 0, sem_i0)
    @pl.when(1 < nd)
    def _():
        _gather(1, 1, sem_i1)
    for j in range(_NCHUNK):
        @pl.when(j < nz)
        def _():
            row = pl.multiple_of(base + floor_t + j * _C, _C)
            pltpu.async_copy(zsh, out.at[pl.ds(row, _C), :], sem_z)

    for i in range(_NCHUNK):
        @pl.when(i < nd)
        def _():
            pltpu.make_async_copy(
                flat.at[gidx.at[i % 3]], bufs[i % 3], isems[i % 3]).wait()
            @pl.when(i == nd - 1)
            def _():
                for j in range(_NCHUNK):
                    @pl.when(j < nz)
                    def _():
                        pltpu.make_async_copy(
                            zsh, out.at[pl.ds(base, _C), :], sem_z).wait()
                _fill_idx(sidx, i % 3, base + _off(i))
                pltpu.async_copy(bufs[i % 3], out.at[sidx.at[i % 3]],
                                 osems[i % 3])
            @pl.when(i != nd - 1)
            def _():
                row = pl.multiple_of(base + i * _C, _C)
                pltpu.async_copy(bufs[i % 3], out.at[pl.ds(row, _C), :],
                                 osems[i % 3])
        @pl.when(i + 2 < nd)
        def _():
            if i >= 1:
                pltpu.make_async_copy(
                    bufs[(i - 1) % 3], out.at[pl.ds(base, _C), :],
                    osems[(i - 1) % 3]).wait()
            _gather(i + 2, (i + 2) % 3, isems[(i + 2) % 3])

    @pl.when(nd >= 3)
    def _():
        pltpu.make_async_copy(buf0, out.at[pl.ds(base, _C), :], sem_o0).wait()
        pltpu.make_async_copy(buf1, out.at[pl.ds(base, _C), :], sem_o1).wait()
        pltpu.make_async_copy(buf2, out.at[pl.ds(base, _C), :], sem_o2).wait()
    @pl.when(nd == 2)
    def _():
        pltpu.make_async_copy(buf0, out.at[pl.ds(base, _C), :], sem_o0).wait()
        pltpu.make_async_copy(buf1, out.at[pl.ds(base, _C), :], sem_o1).wait()
    @pl.when(nd == 1)
    def _():
        pltpu.make_async_copy(buf0, out.at[pl.ds(base, _C), :], sem_o0).wait()
    @pl.when(nd == 0)
    def _():
        for j in range(_NCHUNK):
            @pl.when(j < nz)
            def _():
                pltpu.make_async_copy(
                    zsh, out.at[pl.ds(base, _C), :], sem_z).wait()


def kernel(flat, cu_seqlens):
    cu = cu_seqlens.astype(jnp.int32)
    mesh = plsc.VectorSubcoreMesh(core_axis_name="c", subcore_axis_name="s")
    run = pl.kernel(
        _body,
        out_type=jax.ShapeDtypeStruct((_B * _S, _D), jnp.float32),
        mesh=mesh,
        compiler_params=pltpu.CompilerParams(use_tc_tiling_on_sc=True),
        scratch_types=[
            pltpu.VMEM((40,), jnp.int32),
            pltpu.VMEM_SHARED((_C, _D), jnp.float32),
            pltpu.VMEM((_C, _D), jnp.float32),
            pltpu.VMEM((_C, _D), jnp.float32),
            pltpu.VMEM((_C, _D), jnp.float32),
            pltpu.VMEM((3, _C), jnp.int32),
            pltpu.VMEM((3, _C), jnp.int32),
            pltpu.SemaphoreType.DMA,
            pltpu.SemaphoreType.DMA,
            pltpu.SemaphoreType.DMA,
            pltpu.SemaphoreType.DMA,
            pltpu.SemaphoreType.DMA,
            pltpu.SemaphoreType.DMA,
            pltpu.SemaphoreType.DMA,
            pltpu.SemaphoreType.DMA,
        ],
    )
    out = run(flat, cu)
    return out.reshape(_B, _S, _D)


# speedup vs baseline: 1.0002x; 1.0002x over previous
import jax
import jax.numpy as jnp
from jax import lax
from jax.experimental import pallas as pl
from jax.experimental.pallas import tpu as pltpu
from jax.experimental.pallas import tpu_sc as plsc

_B = 16
_S = 2048
_D = 256
_TOTAL = 16384
_HALF = _S // 2
_C = 128
_NCHUNK = _HALF // _C
_L = 16
_NS = 16
_ZROWS = _C // _NS


def _body(flat, cu, out, cu_v, zsh, buf0, buf1, buf2, gidx, sidx,
          sem_c, sem_z, sem_i0, sem_i1, sem_i2, sem_o0, sem_o1, sem_o2):
    cid = lax.axis_index("c")
    sid = lax.axis_index("s")
    wid = cid * 16 + sid
    b = wid // 2
    h = wid % 2

    pltpu.async_copy(cu, cu_v.at[pl.ds(0, _B + 1)], sem_c)
    pltpu.make_async_copy(cu, cu_v.at[pl.ds(0, _B + 1)], sem_c).wait()
    start = cu_v[pl.ds(b, 16)][0]
    ln = cu_v[pl.ds(b + 1, 16)][0] - start
    t = jnp.clip(ln - h * _HALF, 0, _HALF)
    base = b * _S + h * _HALF
    floor_t = (t // _C) * _C
    nz = (_HALF - floor_t) // _C
    nd = (t + _C - 1) // _C

    lanes = lax.iota(jnp.int32, _L)

    def _off(i):
        return jnp.minimum(i * _C, t - _C)

    def _fill_idx(ref, slot, first_row):
        for k in range(_C // _L):
            ref[slot, pl.ds(k * _L, _L)] = first_row + k * _L + lanes

    def _gather(i, slot, isem):
        _fill_idx(gidx, slot, start + h * _HALF + _off(i))
        pltpu.async_copy(flat.at[gidx.at[slot]], (buf0, buf1, buf2)[slot], isem)

    bufs = (buf0, buf1, buf2)
    isems = (sem_i0, sem_i1, sem_i2)
    osems = (sem_o0, sem_o1, sem_o2)

    @pl.when(0 < nd)
    def _():
        _gather(0, 0, sem_i0)
    @pl.when(1 < nd)
    def _():
        _gather(1, 1, sem_i1)

    def _zero(i, _):
        buf2[i // (_D // _L), pl.ds((i % (_D // _L)) * _L, _L)] = (
            jnp.zeros((_L,), jnp.float32))
        return 0
    lax.fori_loop(0, _ZROWS * (_D // _L), _zero, 0, unroll=8)
    pltpu.sync_copy(buf2.at[pl.ds(0, _ZROWS), :],
                    zsh.at[pl.ds(sid * _ZROWS, _ZROWS), :])
    plsc.subcore_barrier()

    for j in range(_NCHUNK):
        @pl.when(j < nz)
        def _():
            row = pl.multiple_of(base + floor_t + j * _C, _C)
            pltpu.async_copy(zsh, out.at[pl.ds(row, _C), :], sem_z)

    for i in range(_NCHUNK):
        @pl.when(i < nd)
        def _():
            pltpu.make_async_copy(
                flat.at[gidx.at[i % 3]], bufs[i % 3], isems[i % 3]).wait()
            @pl.when(i == nd - 1)
            def _():
                for j in range(_NCHUNK):
                    @pl.when(j < nz)
                    def _():
                        pltpu.make_async_copy(
                            zsh, out.at[pl.ds(base, _C), :], sem_z).wait()
                _fill_idx(sidx, i % 3, base + _off(i))
                pltpu.async_copy(bufs[i % 3], out.at[sidx.at[i % 3]],
                                 osems[i % 3])
            @pl.when(i != nd - 1)
            def _():
                row = pl.multiple_of(base + i * _C, _C)
                pltpu.async_copy(bufs[i % 3], out.at[pl.ds(row, _C), :],
                                 osems[i % 3])
        @pl.when(i + 2 < nd)
        def _():
            if i >= 1:
                pltpu.make_async_copy(
                    bufs[(i - 1) % 3], out.at[pl.ds(base, _C), :],
                    osems[(i - 1) % 3]).wait()
            _gather(i + 2, (i + 2) % 3, isems[(i + 2) % 3])

    @pl.when(nd >= 3)
    def _():
        pltpu.make_async_copy(buf0, out.at[pl.ds(base, _C), :], sem_o0).wait()
        pltpu.make_async_copy(buf1, out.at[pl.ds(base, _C), :], sem_o1).wait()
        pltpu.make_async_copy(buf2, out.at[pl.ds(base, _C), :], sem_o2).wait()
    @pl.when(nd == 2)
    def _():
        pltpu.make_async_copy(buf0, out.at[pl.ds(base, _C), :], sem_o0).wait()
        pltpu.make_async_copy(buf1, out.at[pl.ds(base, _C), :], sem_o1).wait()
    @pl.when(nd == 1)
    def _():
        pltpu.make_async_copy(buf0, out.at[pl.ds(base, _C), :], sem_o0).wait()
    @pl.when(nd == 0)
    def _():
        for j in range(_NCHUNK):
            @pl.when(j < nz)
            def _():
                pltpu.make_async_copy(
                    zsh, out.at[pl.ds(base, _C), :], sem_z).wait()


def kernel(flat, cu_seqlens):
    cu = cu_seqlens.astype(jnp.int32)
    mesh = plsc.VectorSubcoreMesh(core_axis_name="c", subcore_axis_name="s")
    run = pl.kernel(
        _body,
        out_type=jax.ShapeDtypeStruct((_B * _S, _D), jnp.float32),
        mesh=mesh,
        compiler_params=pltpu.CompilerParams(use_tc_tiling_on_sc=True),
        scratch_types=[
            pltpu.VMEM((40,), jnp.int32),
            pltpu.VMEM_SHARED((_C, _D), jnp.float32),
            pltpu.VMEM((_C, _D), jnp.float32),
            pltpu.VMEM((_C, _D), jnp.float32),
            pltpu.VMEM((_C, _D), jnp.float32),
            pltpu.VMEM((3, _C), jnp.int32),
            pltpu.VMEM((3, _C), jnp.int32),
            pltpu.SemaphoreType.DMA,
            pltpu.SemaphoreType.DMA,
            pltpu.SemaphoreType.DMA,
            pltpu.SemaphoreType.DMA,
            pltpu.SemaphoreType.DMA,
            pltpu.SemaphoreType.DMA,
            pltpu.SemaphoreType.DMA,
            pltpu.SemaphoreType.DMA,
        ],
    )
    out = run(flat, cu)
    return out.reshape(_B, _S, _D)


# speedup vs baseline: 1.0017x; 1.0015x over previous
import jax
import jax.numpy as jnp
from jax import lax
from jax.experimental import pallas as pl
from jax.experimental.pallas import tpu as pltpu
from jax.experimental.pallas import tpu_sc as plsc

_B = 16
_S = 2048
_D = 256
_TOTAL = 16384
_HALF = _S // 2
_C = 128
_NCHUNK = _HALF // _C
_L = 16
_NS = 16
_ZROWS = _C // _NS


def _body(flat, cu, out, cu_v, zsh, buf0, buf1, buf2, gidx, sidx,
          sem_c, sem_z, sem_i0, sem_i1, sem_i2, sem_o0, sem_o1, sem_o2):
    cid = lax.axis_index("c")
    sid = lax.axis_index("s")
    wid = cid * 16 + sid
    b = wid // 2
    h = wid % 2

    pltpu.async_copy(cu, cu_v.at[pl.ds(0, _B + 1)], sem_c)
    pltpu.make_async_copy(cu, cu_v.at[pl.ds(0, _B + 1)], sem_c).wait()
    start = cu_v[pl.ds(b, 16)][0]
    ln = cu_v[pl.ds(b + 1, 16)][0] - start
    t = jnp.clip(ln - h * _HALF, 0, _HALF)
    base = b * _S + h * _HALF
    floor_t = (t // _C) * _C
    nz = (_HALF - floor_t) // _C
    nd = (t + _C - 1) // _C

    lanes = lax.iota(jnp.int32, _L)

    def _off(i):
        return jnp.minimum(i * _C, t - _C)

    def _fill_idx(ref, slot, first_row):
        for k in range(_C // _L):
            ref[slot, pl.ds(k * _L, _L)] = first_row + k * _L + lanes

    def _gather(i, slot, isem):
        _fill_idx(gidx, slot, start + h * _HALF + _off(i))
        pltpu.async_copy(flat.at[gidx.at[slot]], (buf0, buf1, buf2)[slot], isem)

    bufs = (buf0, buf1, buf2)
    isems = (sem_i0, sem_i1, sem_i2)
    osems = (sem_o0, sem_o1, sem_o2)

    @pl.when(0 < nd)
    def _():
        _gather(0, 0, sem_i0)
    @pl.when(1 < nd)
    def _():
        _gather(1, 1, sem_i1)

    def _zero(i, _):
        buf2[i // (_D // _L), pl.ds((i % (_D // _L)) * _L, _L)] = (
            jnp.zeros((_L,), jnp.float32))
        return 0
    lax.fori_loop(0, _ZROWS * (_D // _L), _zero, 0, unroll=8)
    pltpu.sync_copy(buf2.at[pl.ds(0, _ZROWS), :],
                    zsh.at[pl.ds(sid * _ZROWS, _ZROWS), :])
    plsc.subcore_barrier()

    for j in range(_NCHUNK):
        @pl.when(j < nz)
        def _():
            row = pl.multiple_of(base + floor_t + j * _C, _C)
            pltpu.async_copy(zsh, out.at[pl.ds(row, _C), :], sem_z)

    for i in range(_NCHUNK):
        @pl.when(i < nd)
        def _():
            pltpu.make_async_copy(
                flat.at[gidx.at[i % 3]], bufs[i % 3], isems[i % 3]).wait()
            @pl.when(i == nd - 1)
            def _():
                for j in range(_NCHUNK):
                    @pl.when(j < nz)
                    def _():
                        pltpu.make_async_copy(
                            zsh, out.at[pl.ds(base, _C), :], sem_z).wait()
                _fill_idx(sidx, i % 3, base + _off(i))
                pltpu.async_copy(bufs[i % 3], out.at[sidx.at[i % 3]],
                                 osems[i % 3])
            @pl.when(i != nd - 1)
            def _():
                row = pl.multiple_of(base + i * _C, _C)
                pltpu.async_copy(bufs[i % 3], out.at[pl.ds(row, _C), :],
                                 osems[i % 3])
        @pl.when(i + 2 < nd)
        def _():
            if i >= 1:
                pltpu.make_async_copy(
                    bufs[(i - 1) % 3], out.at[pl.ds(base, _C), :],
                    osems[(i - 1) % 3]).wait()
            _gather(i + 2, (i + 2) % 3, isems[(i + 2) % 3])

    @pl.when(nd >= 3)
    def _():
        pltpu.make_async_copy(buf0, out.at[pl.ds(base, _C), :], sem_o0).wait()
        pltpu.make_async_copy(buf1, out.at[pl.ds(base, _C), :], sem_o1).wait()
        pltpu.make_async_copy(buf2, out.at[pl.ds(base, _C), :], sem_o2).wait()
    @pl.when(nd == 2)
    def _():
        pltpu.make_async_copy(buf0, out.at[pl.ds(base, _C), :], sem_o0).wait()
        pltpu.make_async_copy(buf1, out.at[pl.ds(base, _C), :], sem_o1).wait()
    @pl.when(nd == 1)
    def _():
        pltpu.make_async_copy(buf0, out.at[pl.ds(base, _C), :], sem_o0).wait()
    @pl.when(nd == 0)
    def _():
        for j in range(_NCHUNK):
            @pl.when(j < nz)
            def _():
                pltpu.make_async_copy(
                    zsh, out.at[pl.ds(base, _C), :], sem_z).wait()


def kernel(flat, cu_seqlens):
    cu = cu_seqlens.astype(jnp.int32)
    mesh = plsc.VectorSubcoreMesh(core_axis_name="c", subcore_axis_name="s")
    run = pl.kernel(
        _body,
        out_type=jax.ShapeDtypeStruct((_B * _S, _D), jnp.float32),
        mesh=mesh,
        compiler_params=pltpu.CompilerParams(
            use_tc_tiling_on_sc=True,
            disable_bounds_checks=True,
            disable_semaphore_checks=True,
            skip_device_barrier=True,
        ),
        scratch_types=[
            pltpu.VMEM((40,), jnp.int32),
            pltpu.VMEM_SHARED((_C, _D), jnp.float32),
            pltpu.VMEM((_C, _D), jnp.float32),
            pltpu.VMEM((_C, _D), jnp.float32),
            pltpu.VMEM((_C, _D), jnp.float32),
            pltpu.VMEM((3, _C), jnp.int32),
            pltpu.VMEM((3, _C), jnp.int32),
            pltpu.SemaphoreType.DMA,
            pltpu.SemaphoreType.DMA,
            pltpu.SemaphoreType.DMA,
            pltpu.SemaphoreType.DMA,
            pltpu.SemaphoreType.DMA,
            pltpu.SemaphoreType.DMA,
            pltpu.SemaphoreType.DMA,
            pltpu.SemaphoreType.DMA,
        ],
    )
    out = run(flat, cu)
    return out.reshape(_B, _S, _D)


# speedup vs baseline: 1.0049x; 1.0032x over previous
import jax
import jax.numpy as jnp
from jax import lax
from jax.experimental import pallas as pl
from jax.experimental.pallas import tpu as pltpu
from jax.experimental.pallas import tpu_sc as plsc

_B = 16
_S = 2048
_D = 256
_TOTAL = 16384
_HALF = _S // 2
_C = 128
_NCHUNK = _HALF // _C
_L = 16
_NS = 16
_ZROWS = _C // _NS


def _body(flat, cu, out, cu_v, zsh, buf0, buf1, buf2, gidx, sidx,
          sem_c, sem_z, sem_i0, sem_i1, sem_i2, sem_o0, sem_o1, sem_o2):
    cid = lax.axis_index("c")
    sid = lax.axis_index("s")
    wid = cid * 16 + sid
    b = wid // 2
    h = wid % 2

    pltpu.async_copy(cu, cu_v.at[pl.ds(0, _B + 1)], sem_c)
    pltpu.make_async_copy(cu, cu_v.at[pl.ds(0, _B + 1)], sem_c).wait()
    start = cu_v[pl.ds(b, 16)][0]
    ln = cu_v[pl.ds(b + 1, 16)][0] - start
    t = jnp.clip(ln - h * _HALF, 0, _HALF)
    base = b * _S + h * _HALF
    floor_t = (t // _C) * _C
    nz = (_HALF - floor_t) // _C
    nd = (t + _C - 1) // _C

    lanes = lax.iota(jnp.int32, _L)

    def _off(i):
        return jnp.minimum(i * _C, t - _C)

    def _fill_idx(ref, slot, first_row):
        for k in range(_C // _L):
            ref[slot, pl.ds(k * _L, _L)] = first_row + k * _L + lanes

    def _gather(i, slot, isem):
        _fill_idx(gidx, slot, start + h * _HALF + _off(i))
        pltpu.async_copy(flat.at[gidx.at[slot]], (buf0, buf1, buf2)[slot], isem)

    bufs = (buf0, buf1, buf2)
    isems = (sem_i0, sem_i1, sem_i2)
    osems = (sem_o0, sem_o1, sem_o2)

    @pl.when(0 < nd)
    def _():
        _gather(0, 0, sem_i0)
    @pl.when(1 < nd)
    def _():
        _gather(1, 1, sem_i1)

    def _zero(i, _):
        buf2[i // (_D // _L), pl.ds((i % (_D // _L)) * _L, _L)] = (
            jnp.zeros((_L,), jnp.float32))
        return 0
    lax.fori_loop(0, _ZROWS * (_D // _L), _zero, 0, unroll=8)
    pltpu.sync_copy(buf2.at[pl.ds(0, _ZROWS), :],
                    zsh.at[pl.ds(sid * _ZROWS, _ZROWS), :])
    plsc.subcore_barrier()

    for j in range(_NCHUNK):
        @pl.when(j < nz)
        def _():
            row = pl.multiple_of(base + floor_t + j * _C, _C)
            pltpu.async_copy(zsh, out.at[pl.ds(row, _C), :], sem_z)

    for i in range(_NCHUNK):
        @pl.when(i < nd)
        def _():
            pltpu.make_async_copy(
                flat.at[gidx.at[i % 3]], bufs[i % 3], isems[i % 3]).wait()
            @pl.when(i == nd - 1)
            def _():
                for j in range(_NCHUNK):
                    @pl.when(j < nz)
                    def _():
                        pltpu.make_async_copy(
                            zsh, out.at[pl.ds(base, _C), :], sem_z).wait()
                _fill_idx(sidx, i % 3, base + _off(i))
                pltpu.async_copy(bufs[i % 3], out.at[sidx.at[i % 3]],
                                 osems[i % 3])
            @pl.when(i != nd - 1)
            def _():
                row = pl.multiple_of(base + i * _C, _C)
                pltpu.async_copy(bufs[i % 3], out.at[pl.ds(row, _C), :],
                                 osems[i % 3])
        @pl.when(i + 2 < nd)
        def _():
            if i >= 1:
                pltpu.make_async_copy(
                    bufs[(i - 1) % 3], out.at[pl.ds(base, _C), :],
                    osems[(i - 1) % 3]).wait()
            _gather(i + 2, (i + 2) % 3, isems[(i + 2) % 3])

    @pl.when(nd >= 3)
    def _():
        pltpu.make_async_copy(buf0, out.at[pl.ds(base, _C), :], sem_o0).wait()
        pltpu.make_async_copy(buf1, out.at[pl.ds(base, _C), :], sem_o1).wait()
        pltpu.make_async_copy(buf2, out.at[pl.ds(base, _C), :], sem_o2).wait()
    @pl.when(nd == 2)
    def _():
        pltpu.make_async_copy(buf0, out.at[pl.ds(base, _C), :], sem_o0).wait()
        pltpu.make_async_copy(buf1, out.at[pl.ds(base, _C), :], sem_o1).wait()
    @pl.when(nd == 1)
    def _():
        pltpu.make_async_copy(buf0, out.at[pl.ds(base, _C), :], sem_o0).wait()
    @pl.when(nd == 0)
    def _():
        for j in range(_NCHUNK):
            @pl.when(j < nz)
            def _():
                pltpu.make_async_copy(
                    zsh, out.at[pl.ds(base, _C), :], sem_z).wait()


def kernel(flat, cu_seqlens):
    cu = cu_seqlens.astype(jnp.int32)
    mesh = plsc.VectorSubcoreMesh(core_axis_name="c", subcore_axis_name="s")
    run = pl.kernel(
        _body,
        out_type=jax.ShapeDtypeStruct((_B * _S, _D), jnp.float32),
        mesh=mesh,
        compiler_params=pltpu.CompilerParams(use_tc_tiling_on_sc=True),
        scratch_types=[
            pltpu.VMEM((40,), jnp.int32),
            pltpu.VMEM_SHARED((_C, _D), jnp.float32),
            pltpu.VMEM((_C, _D), jnp.float32),
            pltpu.VMEM((_C, _D), jnp.float32),
            pltpu.VMEM((_C, _D), jnp.float32),
            pltpu.VMEM((3, _C), jnp.int32),
            pltpu.VMEM((3, _C), jnp.int32),
            pltpu.SemaphoreType.DMA,
            pltpu.SemaphoreType.DMA,
            pltpu.SemaphoreType.DMA,
            pltpu.SemaphoreType.DMA,
            pltpu.SemaphoreType.DMA,
            pltpu.SemaphoreType.DMA,
            pltpu.SemaphoreType.DMA,
            pltpu.SemaphoreType.DMA,
        ],
    )
    out = run(flat, cu)
    return out.reshape(_B, _S, _D)
